# Initial kernel scaffold; baseline (speedup 1.0000x reference)
import jax
import jax.numpy as jnp
from jax.experimental import pallas as pl


def kernel(x_input, edge_index_input, pos_edge_index_input, W1, b1, W2, b2):
    raise NotImplementedError("write your pallas kernel here")


# speedup vs baseline: 5.4273x; 5.4273x over previous
import functools

import jax
import jax.numpy as jnp
from jax import lax
from jax.experimental import pallas as pl
from jax.experimental.pallas import tpu as pltpu
from jax.experimental.pallas import tpu_sc as plsc

NC = 2
NS = 16
NW = NC * NS
L = 16
C = 80


def _mesh():
    return plsc.VectorSubcoreMesh(core_axis_name="c", subcore_axis_name="s",
                                  num_cores=NC, num_subcores=NS)


def _wid():
    return lax.axis_index("s") * NC + lax.axis_index("c")


def _tc_transform(x, W1, b1, W2):
    N, D = x.shape
    BN = 1000

    def body(x_ref, w1_ref, b1_ref, w2_ref, o_ref):
        h = lax.dot_general(x_ref[...], w1_ref[...], (((1,), (1,)), ((), ())),
                            preferred_element_type=jnp.float32) + b1_ref[...]
        o_ref[...] = lax.dot_general(h, w2_ref[...], (((1,), (1,)), ((), ())),
                                     preferred_element_type=jnp.float32)

    return pl.pallas_call(
        body,
        grid=(N // BN,),
        in_specs=[pl.BlockSpec((BN, D), lambda i: (i, 0)),
                  pl.BlockSpec((D, D), lambda i: (0, 0)),
                  pl.BlockSpec((1, D), lambda i: (0, 0)),
                  pl.BlockSpec((D, D), lambda i: (0, 0))],
        out_specs=pl.BlockSpec((BN, D), lambda i: (i, 0)),
        out_shape=jax.ShapeDtypeStruct((N, D), jnp.float32),
    )(x, W1, b1.reshape(1, D), W2)


def _sc_hist(dst, n_nodes):
    (E,) = dst.shape
    EP = E // NW

    @functools.partial(
        pl.kernel,
        out_type=jax.ShapeDtypeStruct((NW * n_nodes,), jnp.float32),
        mesh=_mesh(),
        compiler_params=pltpu.CompilerParams(needs_layout_passes=False),
        scratch_types=[pltpu.VMEM((EP,), jnp.int32),
                       pltpu.VMEM((n_nodes,), jnp.float32)],
    )
    def k(dst_hbm, hist_hbm, didx, hist):
        wid = _wid()
        pltpu.sync_copy(dst_hbm.at[pl.ds(wid * EP, EP)], didx)

        def zero(i, _):
            hist[pl.ds(i * L, L)] = jnp.zeros((L,), jnp.float32)
            return 0
        lax.fori_loop(0, n_nodes // L, zero, 0)

        ones = jnp.ones((L,), jnp.float32)

        def body(i, _):
            idx = didx[pl.ds(i * L, L)]
            plsc.addupdate_scatter(hist, [idx], ones)
            return 0
        lax.fori_loop(0, EP // L, body, 0)
        pltpu.sync_copy(hist, hist_hbm.at[pl.ds(wid * n_nodes, n_nodes)])

    return k(dst)


def _tc_scale(x2, hist_t):
    N, D = x2.shape
    BN = 1000

    def body(x2_ref, h_ref, y_ref):
        deg = jnp.sum(h_ref[...], axis=1, keepdims=True) + 1.0
        y_ref[...] = x2_ref[...] * lax.rsqrt(deg)

    return pl.pallas_call(
        body,
        grid=(N // BN,),
        in_specs=[pl.BlockSpec((BN, D), lambda i: (i, 0)),
                  pl.BlockSpec((BN, NW), lambda i: (i, 0))],
        out_specs=pl.BlockSpec((BN, D), lambda i: (i, 0)),
        out_shape=jax.ShapeDtypeStruct((N, D), jnp.float32),
    )(x2, hist_t)


def _sc_scatter(y, src3d, dst3d):
    N, D = y.shape
    _, CH, _ = src3d.shape
    NPAD = -(-N // (NS * C)) * NS * C
    ZPT = NPAD // (NS * C)
    RPT = NPAD // NS

    @functools.partial(
        pl.kernel,
        out_type=jax.ShapeDtypeStruct((NC, NPAD, D), jnp.float32),
        mesh=_mesh(),
        compiler_params=pltpu.CompilerParams(needs_layout_passes=False),
        scratch_types=[pltpu.VMEM((CH, C), jnp.int32),
                       pltpu.VMEM((CH, C), jnp.int32),
                       pltpu.VMEM((C, D), jnp.float32),
                       pltpu.VMEM_SHARED((NPAD, D), jnp.float32),
                       pltpu.SemaphoreType.DMA],
    )
    def k(y_hbm, src_hbm, dst_hbm, part_hbm, sidx, didx, rowbuf, acc, gsem):
        cid = lax.axis_index("c")
        sid = lax.axis_index("s")
        wid = sid * NC + cid

        pltpu.sync_copy(src_hbm.at[wid], sidx)
        pltpu.sync_copy(dst_hbm.at[wid], didx)

        def zrow(i, _):
            r = i // (D // L)
            c = i % (D // L)
            rowbuf[r, pl.ds(c * L, L)] = jnp.zeros((L,), jnp.float32)
            return 0
        lax.fori_loop(0, C * (D // L), zrow, 0)

        def zacc(z, _):
            pltpu.sync_copy(rowbuf, acc.at[pl.ds((sid * ZPT + z) * C, C)])
            return 0
        lax.fori_loop(0, ZPT, zacc, 0)
        plsc.subcore_barrier()

        def chunk(ck, _):
            pltpu.async_copy(y_hbm.at[sidx.at[ck]], rowbuf, gsem).wait()
            pltpu.sync_copy(rowbuf, acc.at[didx.at[ck]], add=True)
            return 0
        lax.fori_loop(0, CH, chunk, 0)
        plsc.subcore_barrier()

        pltpu.sync_copy(acc.at[pl.ds(sid * RPT, RPT)],
                        part_hbm.at[cid, pl.ds(sid * RPT, RPT)])

    return k(y, src3d, dst3d)


def _tc_finish(parts, y, hist_t, b2):
    N, D = y.shape
    BN = 1000

    def body(p_ref, y_ref, h_ref, b2_ref, o_ref):
        deg = jnp.sum(h_ref[...], axis=1, keepdims=True) + 1.0
        s = p_ref[0] + p_ref[1] + y_ref[...]
        o_ref[...] = lax.rsqrt(deg) * s + b2_ref[...]

    return pl.pallas_call(
        body,
        grid=(N // BN,),
        in_specs=[pl.BlockSpec((NC, BN, D), lambda i: (0, i, 0)),
                  pl.BlockSpec((BN, D), lambda i: (i, 0)),
                  pl.BlockSpec((BN, NW), lambda i: (i, 0)),
                  pl.BlockSpec((1, D), lambda i: (0, 0))],
        out_specs=pl.BlockSpec((BN, D), lambda i: (i, 0)),
        out_shape=jax.ShapeDtypeStruct((N, D), jnp.float32),
    )(parts, y, hist_t, b2.reshape(1, D))


def _sc_logits(out, e0_3d, e1_3d):
    N, D = out.shape
    _, CH, _ = e0_3d.shape
    E = NW * CH * C

    @functools.partial(
        pl.kernel,
        out_type=jax.ShapeDtypeStruct((E,), jnp.float32),
        mesh=_mesh(),
        compiler_params=pltpu.CompilerParams(needs_layout_passes=False),
        scratch_types=[pltpu.VMEM((CH, C), jnp.int32),
                       pltpu.VMEM((CH, C), jnp.int32),
                       pltpu.VMEM((C, D), jnp.float32),
                       pltpu.VMEM((C, D), jnp.float32),
                       pltpu.VMEM((C,), jnp.float32),
                       pltpu.SemaphoreType.DMA,
                       pltpu.SemaphoreType.DMA],
    )
    def k(out_hbm, e0_hbm, e1_hbm, log_hbm, aidx, bidx, abuf, bbuf, res,
          sema, semb):
        wid = _wid()
        pltpu.sync_copy(e0_hbm.at[wid], aidx)
        pltpu.sync_copy(e1_hbm.at[wid], bidx)

        lane = jnp.arange(L, dtype=jnp.int32)

        def chunk(ck, _):
            ca = pltpu.async_copy(out_hbm.at[aidx.at[ck]], abuf, sema)
            cb = pltpu.async_copy(out_hbm.at[bidx.at[ck]], bbuf, semb)
            ca.wait()
            cb.wait()

            def group(g, _):
                rows = lane + g * L
                acc = jnp.zeros((L,), jnp.float32)
                for d in range(D):
                    cols = jnp.full((L,), d, jnp.int32)
                    acc = acc + (plsc.load_gather(abuf, [rows, cols]) *
                                 plsc.load_gather(bbuf, [rows, cols]))
                res[pl.ds(g * L, L)] = acc
                return 0
            lax.fori_loop(0, C // L, group, 0)
            pltpu.sync_copy(res, log_hbm.at[pl.ds((wid * CH + ck) * C, C)])
            return 0
        lax.fori_loop(0, CH, chunk, 0)

    return k(out, e0_3d, e1_3d)


def kernel(x_input, edge_index_input, pos_edge_index_input, W1, b1, W2, b2):
    N, D = x_input.shape
    E = pos_edge_index_input.shape[1]
    CH = E // NW // C
    src = pos_edge_index_input[0].reshape(NW, CH, C)
    dst = pos_edge_index_input[1]
    e0 = edge_index_input[0].reshape(NW, CH, C)
    e1 = edge_index_input[1].reshape(NW, CH, C)

    x2 = _tc_transform(x_input, W1, b1, W2)
    hist = _sc_hist(dst, N)
    hist_t = hist.reshape(NW, N).T
    y = _tc_scale(x2, hist_t)
    parts = _sc_scatter(y, src, dst.reshape(NW, CH, C))
    out = _tc_finish(parts, y, hist_t, b2)
    return _sc_logits(out, e0, e1)


# speedup vs baseline: 6.4347x; 1.1856x over previous
import functools

import jax
import jax.numpy as jnp
from jax import lax
from jax.experimental import pallas as pl
from jax.experimental.pallas import tpu as pltpu
from jax.experimental.pallas import tpu_sc as plsc

NC = 2
NS = 16
NW = NC * NS
L = 16
C4 = 80
C6 = 80
NB = 5


def _mesh():
    return plsc.VectorSubcoreMesh(core_axis_name="c", subcore_axis_name="s",
                                  num_cores=NC, num_subcores=NS)


def _wid():
    return lax.axis_index("s") * NC + lax.axis_index("c")


def _tc_transform(x, W1, b1, W2):
    N, D = x.shape
    BN = 1000

    def body(x_ref, w1_ref, b1_ref, w2_ref, o_ref):
        h = lax.dot_general(x_ref[...], w1_ref[...], (((1,), (1,)), ((), ())),
                            preferred_element_type=jnp.float32) + b1_ref[...]
        o_ref[...] = lax.dot_general(h, w2_ref[...], (((1,), (1,)), ((), ())),
                                     preferred_element_type=jnp.float32)

    return pl.pallas_call(
        body,
        grid=(N // BN,),
        in_specs=[pl.BlockSpec((BN, D), lambda i: (i, 0)),
                  pl.BlockSpec((D, D), lambda i: (0, 0)),
                  pl.BlockSpec((1, D), lambda i: (0, 0)),
                  pl.BlockSpec((D, D), lambda i: (0, 0))],
        out_specs=pl.BlockSpec((BN, D), lambda i: (i, 0)),
        out_shape=jax.ShapeDtypeStruct((N, D), jnp.float32),
    )(x, W1, b1.reshape(1, D), W2)


def _sc_hist(dst, n_nodes):
    (E,) = dst.shape
    EP = E // NW

    @functools.partial(
        pl.kernel,
        out_type=jax.ShapeDtypeStruct((NW * n_nodes,), jnp.float32),
        mesh=_mesh(),
        compiler_params=pltpu.CompilerParams(needs_layout_passes=False),
        scratch_types=[pltpu.VMEM((EP,), jnp.int32),
                       pltpu.VMEM((n_nodes,), jnp.float32)],
    )
    def k(dst_hbm, hist_hbm, didx, hist):
        wid = _wid()
        pltpu.sync_copy(dst_hbm.at[pl.ds(wid * EP, EP)], didx)

        def zero(i, _):
            hist[pl.ds(i * L, L)] = jnp.zeros((L,), jnp.float32)
            return 0
        lax.fori_loop(0, n_nodes // L, zero, 0)

        ones = jnp.ones((L,), jnp.float32)

        def body(i, _):
            idx = didx[pl.ds(i * L, L)]
            plsc.addupdate_scatter(hist, [idx], ones)
            return 0
        lax.fori_loop(0, EP // L, body, 0)
        pltpu.sync_copy(hist, hist_hbm.at[pl.ds(wid * n_nodes, n_nodes)])

    return k(dst)


def _tc_scale(x2, hist_t):
    N, D = x2.shape
    BN = 1000

    def body(x2_ref, h_ref, y_ref):
        deg = jnp.sum(h_ref[...], axis=1, keepdims=True) + 1.0
        y_ref[...] = x2_ref[...] * lax.rsqrt(deg)

    return pl.pallas_call(
        body,
        grid=(N // BN,),
        in_specs=[pl.BlockSpec((BN, D), lambda i: (i, 0)),
                  pl.BlockSpec((BN, NW), lambda i: (i, 0))],
        out_specs=pl.BlockSpec((BN, D), lambda i: (i, 0)),
        out_shape=jax.ShapeDtypeStruct((N, D), jnp.float32),
    )(x2, hist_t)


def _sc_scatter(y, src, dst):
    N, D = y.shape
    (E,) = src.shape
    EP = E // NW
    CH = EP // C4
    NPAD = -(-N // (NS * C4)) * NS * C4
    ZPT = NPAD // (NS * C4)
    RPT = NPAD // NS
    NB4 = 2

    @functools.partial(
        pl.kernel,
        out_type=jax.ShapeDtypeStruct((NC, NPAD, D), jnp.float32),
        mesh=_mesh(),
        compiler_params=pltpu.CompilerParams(needs_layout_passes=False),
        scratch_types=[pltpu.VMEM((EP,), jnp.int32),
                       pltpu.VMEM((EP,), jnp.int32),
                       pltpu.VMEM((NB4, C4, D), jnp.float32),
                       pltpu.VMEM_SHARED((NPAD, D), jnp.float32),
                       pltpu.SemaphoreType.DMA((NB4,)),
                       pltpu.SemaphoreType.DMA((NB4,))],
    )
    def k(y_hbm, src_hbm, dst_hbm, part_hbm, sidx, didx, rowbuf, acc, gsem,
          ssem):
        cid = lax.axis_index("c")
        sid = lax.axis_index("s")
        wid = sid * NC + cid

        pltpu.sync_copy(src_hbm.at[pl.ds(wid * EP, EP)], sidx)
        pltpu.sync_copy(dst_hbm.at[pl.ds(wid * EP, EP)], didx)

        def zrow(i, _):
            r = i // (D // L)
            c = i % (D // L)
            rowbuf[0, r, pl.ds(c * L, L)] = jnp.zeros((L,), jnp.float32)
            return 0
        lax.fori_loop(0, C4 * (D // L), zrow, 0)

        def zacc(z, _):
            pltpu.sync_copy(rowbuf.at[0],
                            acc.at[pl.ds((sid * ZPT + z) * C4, C4)])
            return 0
        lax.fori_loop(0, ZPT, zacc, 0)
        plsc.subcore_barrier()

        def fire(ck, b):
            pltpu.async_copy(y_hbm.at[sidx.at[pl.ds(ck * C4, C4)]],
                             rowbuf.at[b], gsem.at[b])

        def scat(ck, b):
            for j in range(C4 // L):
                vec = didx[pl.ds(ck * C4 + j * L, L)]
                pltpu.async_copy(rowbuf.at[b, pl.ds(j * L, L)], acc.at[vec],
                                 ssem.at[b], add=True)
            for j in range(C4 // L):
                vec = didx[pl.ds(ck * C4 + j * L, L)]
                pltpu.make_async_copy(rowbuf.at[b, pl.ds(j * L, L)],
                                      acc.at[vec], ssem.at[b]).wait()

        def drain(ck, b):
            pltpu.make_async_copy(y_hbm.at[sidx.at[pl.ds(ck * C4, C4)]],
                                  rowbuf.at[b], gsem.at[b]).wait()
            scat(ck, b)

        def prol(b, _):
            fire(b, b)
            return 0
        lax.fori_loop(0, NB4, prol, 0)

        def main(ck, _):
            b = lax.rem(ck, NB4)
            drain(ck, b)
            fire(ck + NB4, b)
            return 0
        lax.fori_loop(0, CH - NB4, main, 0)

        def epi(ck, _):
            drain(ck, lax.rem(ck, NB4))
            return 0
        lax.fori_loop(CH - NB4, CH, epi, 0)
        plsc.subcore_barrier()

        pltpu.sync_copy(acc.at[pl.ds(sid * RPT, RPT)],
                        part_hbm.at[cid, pl.ds(sid * RPT, RPT)])

    return k(y, src, dst)


def _tc_finish(parts, y, hist_t, b2):
    N, D = y.shape
    BN = 1000

    def body(p_ref, y_ref, h_ref, b2_ref, o_ref):
        deg = jnp.sum(h_ref[...], axis=1, keepdims=True) + 1.0
        s = p_ref[0] + p_ref[1] + y_ref[...]
        o_ref[...] = lax.rsqrt(deg) * s + b2_ref[...]

    return pl.pallas_call(
        body,
        grid=(N // BN,),
        in_specs=[pl.BlockSpec((NC, BN, D), lambda i: (0, i, 0)),
                  pl.BlockSpec((BN, D), lambda i: (i, 0)),
                  pl.BlockSpec((BN, NW), lambda i: (i, 0)),
                  pl.BlockSpec((1, D), lambda i: (0, 0))],
        out_specs=pl.BlockSpec((BN, D), lambda i: (i, 0)),
        out_shape=jax.ShapeDtypeStruct((N, D), jnp.float32),
    )(parts, y, hist_t, b2.reshape(1, D))


def _sc_logits(out, e0, e1):
    N, D = out.shape
    (E,) = e0.shape
    EP = E // NW
    CH = EP // C6

    @functools.partial(
        pl.kernel,
        out_type=jax.ShapeDtypeStruct((E,), jnp.float32),
        mesh=_mesh(),
        compiler_params=pltpu.CompilerParams(needs_layout_passes=False),
        scratch_types=[pltpu.VMEM((EP,), jnp.int32),
                       pltpu.VMEM((EP,), jnp.int32),
                       pltpu.VMEM((NB, C6, D), jnp.float32),
                       pltpu.VMEM((NB, C6, D), jnp.float32),
                       pltpu.VMEM((NB, C6), jnp.float32),
                       pltpu.SemaphoreType.DMA((NB,)),
                       pltpu.SemaphoreType.DMA((NB,)),
                       pltpu.SemaphoreType.DMA((NB,))],
    )
    def k(out_hbm, e0_hbm, e1_hbm, log_hbm, aidx, bidx, abuf, bbuf, res,
          sema, semb, semr):
        wid = _wid()
        pltpu.sync_copy(e0_hbm.at[pl.ds(wid * EP, EP)], aidx)
        pltpu.sync_copy(e1_hbm.at[pl.ds(wid * EP, EP)], bidx)

        lane = jnp.arange(L, dtype=jnp.int32)

        def fire(ck, b):
            pltpu.async_copy(out_hbm.at[aidx.at[pl.ds(ck * C6, C6)]],
                             abuf.at[b], sema.at[b])
            pltpu.async_copy(out_hbm.at[bidx.at[pl.ds(ck * C6, C6)]],
                             bbuf.at[b], semb.at[b])

        def rstore_wait(ck, b):
            pltpu.make_async_copy(res.at[b],
                                  log_hbm.at[pl.ds((wid * CH + ck) * C6, C6)],
                                  semr.at[b]).wait()

        def drain(ck, b):
            pltpu.make_async_copy(out_hbm.at[aidx.at[pl.ds(ck * C6, C6)]],
                                  abuf.at[b], sema.at[b]).wait()
            pltpu.make_async_copy(out_hbm.at[bidx.at[pl.ds(ck * C6, C6)]],
                                  bbuf.at[b], semb.at[b]).wait()

            def group(grp, _):
                rows = lane + grp * L
                acc = jnp.zeros((L,), jnp.float32)
                for d in range(D):
                    cols = jnp.full((L,), d, jnp.int32)
                    acc = acc + (plsc.load_gather(abuf.at[b], [rows, cols]) *
                                 plsc.load_gather(bbuf.at[b], [rows, cols]))
                res[b, pl.ds(grp * L, L)] = acc
                return 0
            lax.fori_loop(0, C6 // L, group, 0)
            pltpu.async_copy(res.at[b],
                             log_hbm.at[pl.ds((wid * CH + ck) * C6, C6)],
                             semr.at[b])

        def prol(b, _):
            fire(b, b)
            return 0
        lax.fori_loop(0, NB, prol, 0)

        def main(ck, _):
            b = lax.rem(ck, NB)
            pl.when(ck >= NB)(lambda: rstore_wait(ck - NB, b))
            drain(ck, b)
            fire(ck + NB, b)
            return 0
        lax.fori_loop(0, CH - NB, main, 0)

        def epi(ck, _):
            b = lax.rem(ck, NB)
            pl.when(ck >= NB)(lambda: rstore_wait(ck - NB, b))
            drain(ck, b)
            return 0
        lax.fori_loop(CH - NB, CH, epi, 0)

        def fdrain(i, _):
            ck = CH - NB + i
            rstore_wait(ck, lax.rem(ck, NB))
            return 0
        lax.fori_loop(0, NB, fdrain, 0)

    return k(out, e0, e1)


def kernel(x_input, edge_index_input, pos_edge_index_input, W1, b1, W2, b2):
    N, D = x_input.shape
    src = pos_edge_index_input[0]
    dst = pos_edge_index_input[1]
    e0 = edge_index_input[0]
    e1 = edge_index_input[1]

    x2 = _tc_transform(x_input, W1, b1, W2)
    hist = _sc_hist(dst, N)
    hist_t = hist.reshape(NW, N).T
    y = _tc_scale(x2, hist_t)
    parts = _sc_scatter(y, src, dst)
    out = _tc_finish(parts, y, hist_t, b2)
    return _sc_logits(out, e0, e1)
